# baseline (speedup 1.0000x reference)
import functools

import jax
import jax.numpy as jnp
from jax import lax
from jax.experimental import pallas as pl
from jax.experimental.pallas import tpu as pltpu
from jax.experimental.pallas import tpu_sc as plsc

_B = 128
_CH = 48
_THW = 4096
_NCORES = 2
_NSUB = 16
_CH_PER_SUB = _CH // _NSUB
_B_PER_CORE = _B // _NCORES
_WINDOW = 8


def _sc_body(col_hbm, row_hbm, temp_hbm, out_hbm, colv, rowv, tempv, patv, sem):
    c = lax.axis_index("c")
    s = lax.axis_index("s")

    pltpu.sync_copy(col_hbm, colv)
    pltpu.sync_copy(row_hbm, rowv)
    pltpu.sync_copy(temp_hbm, tempv)

    lane = lax.iota(jnp.int32, 16)

    for k in range(_CH_PER_SUB):
        ch = s * _CH_PER_SUB + k
        ch_vec = jnp.full((16,), ch, jnp.int32)
        is_a = ch_vec < 16
        is_b = ch_vec < 32
        cha = jnp.full((16,), jnp.minimum(ch, 15), jnp.int32)
        chb = jnp.full((16,), jnp.clip(ch - 16, 0, 15), jnp.int32)
        chc = jnp.full((16,), jnp.clip(ch - 32, 0, 15), jnp.int32)
        period = []
        for j in range(16):
            a = plsc.load_gather(colv, [jnp.full((16,), 16 * j, jnp.int32) + cha])
            bv = plsc.load_gather(rowv, [lane * 16 + chb])
            cv = plsc.load_gather(tempv, [chc * 16 + lane])
            v = jnp.where(is_a, a, jnp.where(is_b, bv, cv))
            period.append(v)
        for rep in range(16):
            for j in range(16):
                patv[pl.ds(_THW * k + 256 * rep + 16 * j, 16)] = period[j]

    slice_len = _CH_PER_SUB * _THW
    b0 = c * _B_PER_CORE
    base = s * slice_len

    def issue(i, carry):
        off = (b0 + i) * _CH * _THW + base
        pltpu.make_async_copy(patv, out_hbm.at[pl.ds(off, slice_len)], sem).start()

        @pl.when(i >= _WINDOW)
        def _():
            prev = (b0 + i - _WINDOW) * _CH * _THW + base
            pltpu.make_async_copy(
                patv, out_hbm.at[pl.ds(prev, slice_len)], sem).wait()

        return carry

    lax.fori_loop(0, _B_PER_CORE, issue, 0)

    def drain(i, carry):
        off = (b0 + i) * _CH * _THW + base
        pltpu.make_async_copy(patv, out_hbm.at[pl.ds(off, slice_len)], sem).wait()
        return carry

    lax.fori_loop(_B_PER_CORE - _WINDOW, _B_PER_CORE, drain, 0)


@functools.partial(jax.jit, static_argnums=())
def _sc_call(col16, row16, temp16):
    mesh = plsc.VectorSubcoreMesh(core_axis_name="c", subcore_axis_name="s")
    f = pl.kernel(
        _sc_body,
        out_type=jax.ShapeDtypeStruct((_B * _CH * _THW,), jnp.float32),
        mesh=mesh,
        scratch_types=[
            pltpu.VMEM((256,), jnp.float32),
            pltpu.VMEM((256,), jnp.float32),
            pltpu.VMEM((256,), jnp.float32),
            pltpu.VMEM((_CH_PER_SUB * _THW,), jnp.float32),
            pltpu.SemaphoreType.DMA,
        ],
        compiler_params=pltpu.CompilerParams(needs_layout_passes=False),
    )
    return f(col16, row16, temp16)


def kernel(x, row_embed, col_embed, temp_embed):
    b, d, t, h, w = x.shape
    f = row_embed.shape[1]
    out_flat = _sc_call(
        col_embed[:h].reshape(-1),
        row_embed[:w].reshape(-1),
        temp_embed[:t].reshape(-1),
    )
    return out_flat.reshape(b, 3 * f, t, h, w)


# speedup vs baseline: 5.5020x; 5.5020x over previous
import functools

import jax
import jax.numpy as jnp
from jax import lax
from jax.experimental import pallas as pl
from jax.experimental.pallas import tpu as pltpu
from jax.experimental.pallas import tpu_sc as plsc

_B = 128
_CH = 48
_F = 16
_THW = 4096
_PERIOD = 256
_NCORES = 2
_NSUB = 16
_B_PER_CORE = _B // _NCORES
_B_PER_SUB = _B_PER_CORE // _NSUB


def _sc_body(col_hbm, row_hbm, temp_hbm, out_hbm, colv, rowv, tempv, patv,
             shared, sem):
    c = lax.axis_index("c")
    s = lax.axis_index("s")

    pltpu.sync_copy(col_hbm, colv)
    pltpu.sync_copy(row_hbm, rowv)
    pltpu.sync_copy(temp_hbm, tempv)

    lane = lax.iota(jnp.int32, 16)

    for ch in range(_CH):
        if ch < _F:
            for j in range(16):
                row = colv[pl.ds(16 * j, 16)]
                patv[ch, pl.ds(16 * j, 16)] = jnp.full(
                    (16,), row[ch], jnp.float32)
        elif ch < 2 * _F:
            v = plsc.load_gather(rowv, [lane * 16 + (ch - _F)])
            for j in range(16):
                patv[ch, pl.ds(16 * j, 16)] = v
        else:
            v = tempv[pl.ds(16 * (ch - 2 * _F), 16)]
            for j in range(16):
                patv[ch, pl.ds(16 * j, 16)] = v

    pltpu.sync_copy(patv, shared.at[:, pl.ds(_PERIOD * s, _PERIOD)])
    plsc.subcore_barrier()

    b0 = c * _B_PER_CORE + s * _B_PER_SUB
    for i in range(_B_PER_SUB):
        pltpu.make_async_copy(shared, out_hbm.at[b0 + i], sem).start()
    for i in range(_B_PER_SUB):
        pltpu.make_async_copy(shared, out_hbm.at[b0 + i], sem).wait()


@functools.partial(jax.jit, static_argnums=())
def _sc_call(col16, row16, temp16):
    mesh = plsc.VectorSubcoreMesh(core_axis_name="c", subcore_axis_name="s")
    f = pl.kernel(
        _sc_body,
        out_type=jax.ShapeDtypeStruct((_B, _CH, _THW), jnp.float32),
        mesh=mesh,
        scratch_types=[
            pltpu.VMEM((256,), jnp.float32),
            pltpu.VMEM((256,), jnp.float32),
            pltpu.VMEM((256,), jnp.float32),
            pltpu.VMEM((_CH, _PERIOD), jnp.float32),
            pltpu.VMEM_SHARED((_CH, _THW), jnp.float32),
            pltpu.SemaphoreType.DMA,
        ],
        compiler_params=pltpu.CompilerParams(needs_layout_passes=False),
    )
    return f(col16, row16, temp16)


def kernel(x, row_embed, col_embed, temp_embed):
    b, d, t, h, w = x.shape
    f = row_embed.shape[1]
    out_flat = _sc_call(
        col_embed[:h].reshape(-1),
        row_embed[:w].reshape(-1),
        temp_embed[:t].reshape(-1),
    )
    return out_flat.reshape(b, 3 * f, t, h, w)


# speedup vs baseline: 6.3394x; 1.1522x over previous
import functools

import jax
import jax.numpy as jnp
from jax import lax
from jax.experimental import pallas as pl
from jax.experimental.pallas import tpu as pltpu
from jax.experimental.pallas import tpu_sc as plsc

_B = 128
_CH = 48
_F = 16
_THW = 4096
_PERIOD = 256
_NCORES = 2
_NSUB = 16
_B_PER_CORE = _B // _NCORES
_WINDOW = 8


def _sc_body(col_hbm, row_hbm, temp_hbm, out_hbm, colv, rowv, tempv, patv, sem):
    c = lax.axis_index("c")
    s = lax.axis_index("s")

    pltpu.sync_copy(col_hbm, colv)
    pltpu.sync_copy(row_hbm, rowv)
    pltpu.sync_copy(temp_hbm, tempv)

    lane = lax.iota(jnp.int32, 16)

    for ch in range(_CH):
        if ch < _F:
            for j in range(16):
                row = colv[pl.ds(16 * j, 16)]
                patv[ch, pl.ds(16 * j, 16)] = jnp.full(
                    (16,), row[ch], jnp.float32)
        elif ch < 2 * _F:
            v = plsc.load_gather(rowv, [lane * 16 + (ch - _F)])
            for j in range(16):
                patv[ch, pl.ds(16 * j, 16)] = v
        else:
            v = tempv[pl.ds(16 * (ch - 2 * _F), 16)]
            for j in range(16):
                patv[ch, pl.ds(16 * j, 16)] = v

    b0 = c * _B_PER_CORE
    lo = s * _PERIOD

    def issue(i, carry):
        dst = out_hbm.at[b0 + i, :, pl.ds(lo, _PERIOD)]
        pltpu.make_async_copy(patv, dst, sem).start()

        @pl.when(i >= _WINDOW)
        def _():
            prev = out_hbm.at[b0 + i - _WINDOW, :, pl.ds(lo, _PERIOD)]
            pltpu.make_async_copy(patv, prev, sem).wait()

        return carry

    lax.fori_loop(0, _B_PER_CORE, issue, 0)

    def drain(i, carry):
        dst = out_hbm.at[b0 + i, :, pl.ds(lo, _PERIOD)]
        pltpu.make_async_copy(patv, dst, sem).wait()
        return carry

    lax.fori_loop(_B_PER_CORE - _WINDOW, _B_PER_CORE, drain, 0)


@functools.partial(jax.jit, static_argnums=())
def _sc_call(col16, row16, temp16):
    mesh = plsc.VectorSubcoreMesh(core_axis_name="c", subcore_axis_name="s")
    f = pl.kernel(
        _sc_body,
        out_type=jax.ShapeDtypeStruct((_B, _CH, _THW), jnp.float32),
        mesh=mesh,
        scratch_types=[
            pltpu.VMEM((256,), jnp.float32),
            pltpu.VMEM((256,), jnp.float32),
            pltpu.VMEM((256,), jnp.float32),
            pltpu.VMEM((_CH, _PERIOD), jnp.float32),
            pltpu.SemaphoreType.DMA,
        ],
        compiler_params=pltpu.CompilerParams(needs_layout_passes=False),
    )
    return f(col16, row16, temp16)


def kernel(x, row_embed, col_embed, temp_embed):
    b, d, t, h, w = x.shape
    f = row_embed.shape[1]
    out_flat = _sc_call(
        col_embed[:h].reshape(-1),
        row_embed[:w].reshape(-1),
        temp_embed[:t].reshape(-1),
    )
    return out_flat.reshape(b, 3 * f, t, h, w)
